# baseline (speedup 1.0000x reference)
import functools

import jax
import jax.numpy as jnp
from jax import lax
from jax.experimental import pallas as pl
from jax.experimental.pallas import tpu as pltpu
from jax.experimental.pallas import tpu_sc as plsc

_N_NODES = 50000
_N_EDGES = 1600000
_AVG_LOG_DEGREE = 3.4965

_NC = 2
_NS = 16
_NW = _NC * _NS
_CHUNK = 49920
_CH = 2080
_N_DESC = _CHUNK // _CH
_NBUF = 8
_NSTAGE = 6
_SCHUNK = _CHUNK // _NSTAGE
_DESC_PER_STAGE = _N_DESC // _NSTAGE
_EXTRA = 128
_EXTRA_BASE = _NW * _CHUNK
_N_EXTRA = (_N_EDGES - _EXTRA_BASE) // _EXTRA
_N_PAD = 50176
_SLICE = _N_PAD // _NS


def _sc_hist_body(ei_hbm, out_hbm, idx_v, extra_v, ones_v, zeros_v, sum_v,
                  b0, b1, b2, b3, b4, b5, b6, b7, hist_sh,
                  s0, s1, s2, s3, s4, s5, s6, s7, t0, t1, t2, t3, t4, t5):
    c = lax.axis_index("c")
    s = lax.axis_index("s")
    w = c * _NS + s
    bufs = (b0, b1, b2, b3, b4, b5, b6, b7)
    sems = (s0, s1, s2, s3, s4, s5, s6, s7)
    stage_sems = (t0, t1, t2, t3, t4, t5)

    base = pl.multiple_of(w * _CHUNK, 128)
    for k in range(_NSTAGE):
        pltpu.async_copy(
            ei_hbm.at[pl.ds(0, 2), pl.ds(base + k * _SCHUNK, _SCHUNK)],
            idx_v.at[pl.ds(0, 2), pl.ds(k * _SCHUNK, _SCHUNK)],
            stage_sems[k],
        )

    ones16 = jnp.ones((16,), jnp.float32)
    zeros16 = jnp.zeros((16,), jnp.float32)

    def _oinit(i):
        ones_v[pl.ds(i * 16, 16)] = ones16

    plsc.parallel_loop(0, _CH // 16, unroll=8)(_oinit)

    def _zinit(i):
        zeros_v[pl.ds(i * 16, 16)] = zeros16

    plsc.parallel_loop(0, _SLICE // 16, unroll=8)(_zinit)

    off = pl.multiple_of(s * _SLICE, 8)
    pltpu.sync_copy(zeros_v, hist_sh.at[pl.ds(off, _SLICE)])
    plsc.subcore_barrier()

    @pl.when(w < _N_EXTRA)
    def _():
        eoff = pl.multiple_of(_EXTRA_BASE + w * _EXTRA, 128)
        pltpu.sync_copy(ei_hbm.at[pl.ds(0, 2), pl.ds(eoff, _EXTRA)], extra_v)

        def _erp(i, carry):
            b0[pl.ds(i * 16, 16)] = extra_v[1, pl.ds(i * 16, 16)]
            return carry

        lax.fori_loop(0, _EXTRA // 16, _erp, 0)
        cp = pltpu.async_copy(
            ones_v.at[pl.ds(0, _EXTRA)], hist_sh.at[b0.at[pl.ds(0, _EXTRA)]],
            s0, add=True,
        )
        cp.wait()

    for j in range(_N_DESC):
        if j % _DESC_PER_STAGE == 0:
            k = j // _DESC_PER_STAGE
            pltpu.make_async_copy(
                ei_hbm.at[pl.ds(0, 2), pl.ds(base + k * _SCHUNK, _SCHUNK)],
                idx_v.at[pl.ds(0, 2), pl.ds(k * _SCHUNK, _SCHUNK)],
                stage_sems[k],
            ).wait()
        buf = bufs[j % _NBUF]
        sem = sems[j % _NBUF]
        if j >= _NBUF:
            pltpu.make_async_copy(ones_v, hist_sh.at[buf], sem).wait()

        def _rp(i, _buf=buf, _j=j):
            _buf[pl.ds(i * 16, 16)] = idx_v[1, pl.ds(_j * _CH + i * 16, 16)]

        plsc.parallel_loop(0, _CH // 16, unroll=8)(_rp)
        pltpu.async_copy(ones_v, hist_sh.at[buf], sem, add=True)

    for j in range(_N_DESC - _NBUF, _N_DESC):
        pltpu.make_async_copy(
            ones_v, hist_sh.at[bufs[j % _NBUF]], sems[j % _NBUF]
        ).wait()

    plsc.subcore_barrier()

    oout = pl.multiple_of(c * _N_PAD + s * _SLICE, 8)
    pltpu.sync_copy(hist_sh.at[pl.ds(off, _SLICE)], sum_v)
    pltpu.sync_copy(sum_v, out_hbm.at[pl.ds(oout, _SLICE)])


@functools.cache
def _sc_hist():
    return pl.kernel(
        _sc_hist_body,
        out_type=jax.ShapeDtypeStruct((_NC * _N_PAD,), jnp.float32),
        mesh=plsc.VectorSubcoreMesh(core_axis_name="c", subcore_axis_name="s"),
        scratch_types=[
            pltpu.VMEM((2, _CHUNK), jnp.int32),
            pltpu.VMEM((2, _EXTRA), jnp.int32),
            pltpu.VMEM((_CH,), jnp.float32),
            pltpu.VMEM((_SLICE,), jnp.float32),
            pltpu.VMEM((_SLICE,), jnp.float32),
            pltpu.VMEM((_CH,), jnp.int32),
            pltpu.VMEM((_CH,), jnp.int32),
            pltpu.VMEM((_CH,), jnp.int32),
            pltpu.VMEM((_CH,), jnp.int32),
            pltpu.VMEM((_CH,), jnp.int32),
            pltpu.VMEM((_CH,), jnp.int32),
            pltpu.VMEM((_CH,), jnp.int32),
            pltpu.VMEM((_CH,), jnp.int32),
            pltpu.MemorySpace.VMEM_SHARED((_N_PAD,), jnp.float32),
            pltpu.SemaphoreType.DMA,
            pltpu.SemaphoreType.DMA,
            pltpu.SemaphoreType.DMA,
            pltpu.SemaphoreType.DMA,
            pltpu.SemaphoreType.DMA,
            pltpu.SemaphoreType.DMA,
            pltpu.SemaphoreType.DMA,
            pltpu.SemaphoreType.DMA,
            pltpu.SemaphoreType.DMA,
            pltpu.SemaphoreType.DMA,
            pltpu.SemaphoreType.DMA,
            pltpu.SemaphoreType.DMA,
            pltpu.SemaphoreType.DMA,
            pltpu.SemaphoreType.DMA,
        ],
    )


def _tc_finish_body(alpha_ref, part_ref, out_ref):
    h = part_ref[0] + part_ref[1]
    a = alpha_ref[0, 0]
    out_ref[...] = (jnp.log(h + 1.0) / _AVG_LOG_DEGREE) ** a


_tc_finish = pl.pallas_call(
    _tc_finish_body,
    out_shape=jax.ShapeDtypeStruct((_N_PAD // 128, 128), jnp.float32),
    in_specs=[
        pl.BlockSpec(memory_space=pltpu.MemorySpace.SMEM),
        pl.BlockSpec(memory_space=pltpu.MemorySpace.VMEM),
    ],
    out_specs=pl.BlockSpec(memory_space=pltpu.MemorySpace.VMEM),
)


def kernel(edge_index, alpha):
    partial = _sc_hist()(edge_index)
    part3d = partial.reshape(_NC, _N_PAD // 128, 128)
    s = _tc_finish(alpha.reshape(1, 1), part3d)
    return s.reshape(_N_PAD)[:_N_NODES]


# speedup vs baseline: 1.0135x; 1.0135x over previous
import functools

import jax
import jax.numpy as jnp
from jax import lax
from jax.experimental import pallas as pl
from jax.experimental.pallas import tpu as pltpu
from jax.experimental.pallas import tpu_sc as plsc

_N_NODES = 50000
_N_EDGES = 1600000
_AVG_LOG_DEGREE = 3.4965

_NC = 2
_NS = 16
_NW = _NC * _NS
_CHUNK = 49920
_CH = 4160
_N_DESC = _CHUNK // _CH
_NBUF = 4
_NSTAGE = 6
_SCHUNK = _CHUNK // _NSTAGE
_DESC_PER_STAGE = _N_DESC // _NSTAGE
_EXTRA = 128
_EXTRA_BASE = _NW * _CHUNK
_N_EXTRA = (_N_EDGES - _EXTRA_BASE) // _EXTRA
_N_PAD = 50176
_SLICE = _N_PAD // _NS


def _sc_hist_body(ei_hbm, out_hbm, idx_v, extra_v, ones_v, zeros_v, sum_v,
                  b0, b1, b2, b3, hist_sh,
                  s0, s1, s2, s3, t0, t1, t2, t3, t4, t5):
    c = lax.axis_index("c")
    s = lax.axis_index("s")
    w = c * _NS + s
    bufs = (b0, b1, b2, b3)
    sems = (s0, s1, s2, s3)
    stage_sems = (t0, t1, t2, t3, t4, t5)

    base = pl.multiple_of(w * _CHUNK, 128)
    for k in range(_NSTAGE):
        pltpu.async_copy(
            ei_hbm.at[pl.ds(0, 2), pl.ds(base + k * _SCHUNK, _SCHUNK)],
            idx_v.at[pl.ds(0, 2), pl.ds(k * _SCHUNK, _SCHUNK)],
            stage_sems[k],
        )

    ones16 = jnp.ones((16,), jnp.float32)
    zeros16 = jnp.zeros((16,), jnp.float32)

    def _oinit(i):
        ones_v[pl.ds(i * 16, 16)] = ones16

    plsc.parallel_loop(0, _CH // 16, unroll=8)(_oinit)

    def _zinit(i):
        zeros_v[pl.ds(i * 16, 16)] = zeros16

    plsc.parallel_loop(0, _SLICE // 16, unroll=8)(_zinit)

    off = pl.multiple_of(s * _SLICE, 8)
    pltpu.sync_copy(zeros_v, hist_sh.at[pl.ds(off, _SLICE)])
    plsc.subcore_barrier()

    @pl.when(w < _N_EXTRA)
    def _():
        eoff = pl.multiple_of(_EXTRA_BASE + w * _EXTRA, 128)
        pltpu.sync_copy(ei_hbm.at[pl.ds(0, 2), pl.ds(eoff, _EXTRA)], extra_v)

        def _erp(i, carry):
            b0[pl.ds(i * 16, 16)] = extra_v[1, pl.ds(i * 16, 16)]
            return carry

        lax.fori_loop(0, _EXTRA // 16, _erp, 0)
        cp = pltpu.async_copy(
            ones_v.at[pl.ds(0, _EXTRA)], hist_sh.at[b0.at[pl.ds(0, _EXTRA)]],
            s0, add=True,
        )
        cp.wait()

    for j in range(_N_DESC):
        if j % _DESC_PER_STAGE == 0:
            k = j // _DESC_PER_STAGE
            pltpu.make_async_copy(
                ei_hbm.at[pl.ds(0, 2), pl.ds(base + k * _SCHUNK, _SCHUNK)],
                idx_v.at[pl.ds(0, 2), pl.ds(k * _SCHUNK, _SCHUNK)],
                stage_sems[k],
            ).wait()
        buf = bufs[j % _NBUF]
        sem = sems[j % _NBUF]
        if j >= _NBUF:
            pltpu.make_async_copy(ones_v, hist_sh.at[buf], sem).wait()

        def _rp(i, _buf=buf, _j=j):
            _buf[pl.ds(i * 16, 16)] = idx_v[1, pl.ds(_j * _CH + i * 16, 16)]

        plsc.parallel_loop(0, _CH // 16, unroll=8)(_rp)
        pltpu.async_copy(ones_v, hist_sh.at[buf], sem, add=True)

    for j in range(_N_DESC - _NBUF, _N_DESC):
        pltpu.make_async_copy(
            ones_v, hist_sh.at[bufs[j % _NBUF]], sems[j % _NBUF]
        ).wait()

    plsc.subcore_barrier()

    oout = pl.multiple_of(c * _N_PAD + s * _SLICE, 8)
    pltpu.sync_copy(hist_sh.at[pl.ds(off, _SLICE)], sum_v)
    pltpu.sync_copy(sum_v, out_hbm.at[pl.ds(oout, _SLICE)])


@functools.cache
def _sc_hist():
    return pl.kernel(
        _sc_hist_body,
        out_type=jax.ShapeDtypeStruct((_NC * _N_PAD,), jnp.float32),
        mesh=plsc.VectorSubcoreMesh(core_axis_name="c", subcore_axis_name="s"),
        scratch_types=[
            pltpu.VMEM((2, _CHUNK), jnp.int32),
            pltpu.VMEM((2, _EXTRA), jnp.int32),
            pltpu.VMEM((_CH,), jnp.float32),
            pltpu.VMEM((_SLICE,), jnp.float32),
            pltpu.VMEM((_SLICE,), jnp.float32),
            pltpu.VMEM((_CH,), jnp.int32),
            pltpu.VMEM((_CH,), jnp.int32),
            pltpu.VMEM((_CH,), jnp.int32),
            pltpu.VMEM((_CH,), jnp.int32),
            pltpu.MemorySpace.VMEM_SHARED((_N_PAD,), jnp.float32),
            pltpu.SemaphoreType.DMA,
            pltpu.SemaphoreType.DMA,
            pltpu.SemaphoreType.DMA,
            pltpu.SemaphoreType.DMA,
            pltpu.SemaphoreType.DMA,
            pltpu.SemaphoreType.DMA,
            pltpu.SemaphoreType.DMA,
            pltpu.SemaphoreType.DMA,
            pltpu.SemaphoreType.DMA,
            pltpu.SemaphoreType.DMA,
        ],
    )


def _tc_finish_body(alpha_ref, part_ref, out_ref):
    h = part_ref[0] + part_ref[1]
    a = alpha_ref[0, 0]
    out_ref[...] = (jnp.log(h + 1.0) / _AVG_LOG_DEGREE) ** a


_tc_finish = pl.pallas_call(
    _tc_finish_body,
    out_shape=jax.ShapeDtypeStruct((_N_PAD // 128, 128), jnp.float32),
    in_specs=[
        pl.BlockSpec(memory_space=pltpu.MemorySpace.SMEM),
        pl.BlockSpec(memory_space=pltpu.MemorySpace.VMEM),
    ],
    out_specs=pl.BlockSpec(memory_space=pltpu.MemorySpace.VMEM),
)


def kernel(edge_index, alpha):
    partial = _sc_hist()(edge_index)
    part3d = partial.reshape(_NC, _N_PAD // 128, 128)
    s = _tc_finish(alpha.reshape(1, 1), part3d)
    return s.reshape(_N_PAD)[:_N_NODES]


# speedup vs baseline: 1.0172x; 1.0036x over previous
import functools

import jax
import jax.numpy as jnp
from jax import lax
from jax.experimental import pallas as pl
from jax.experimental.pallas import tpu as pltpu
from jax.experimental.pallas import tpu_sc as plsc

_N_NODES = 50000
_N_EDGES = 1600000
_AVG_LOG_DEGREE = 3.4965

_NC = 2
_NS = 16
_NW = _NC * _NS
_CHUNK = 49920
_CH = 4160
_N_DESC = _CHUNK // _CH
_NBUF = 4
_NSTAGE = 6
_SCHUNK = _CHUNK // _NSTAGE
_DESC_PER_STAGE = _N_DESC // _NSTAGE
_EXTRA = 128
_EXTRA_BASE = _NW * _CHUNK
_N_EXTRA = (_N_EDGES - _EXTRA_BASE) // _EXTRA
_N_PAD = 50176
_SLICE = _N_PAD // _NS


def _sc_hist_body(ei_hbm, out_hbm, idx_v, extra_v, ones_v, zeros_v, sum_v,
                  b0, b1, b2, b3, ebuf, hist_sh,
                  s0, s1, s2, s3, e0, t0, t1, t2, t3, t4, t5):
    c = lax.axis_index("c")
    s = lax.axis_index("s")
    w = c * _NS + s
    bufs = (b0, b1, b2, b3)
    sems = (s0, s1, s2, s3)
    stage_sems = (t0, t1, t2, t3, t4, t5)

    base = pl.multiple_of(w * _CHUNK, 128)
    for k in range(_NSTAGE):
        pltpu.async_copy(
            ei_hbm.at[pl.ds(0, 2), pl.ds(base + k * _SCHUNK, _SCHUNK)],
            idx_v.at[pl.ds(0, 2), pl.ds(k * _SCHUNK, _SCHUNK)],
            stage_sems[k],
        )

    ones16 = jnp.ones((16,), jnp.float32)
    zeros16 = jnp.zeros((16,), jnp.float32)

    def _oinit(i):
        ones_v[pl.ds(i * 16, 16)] = ones16

    plsc.parallel_loop(0, _CH // 16, unroll=8)(_oinit)

    def _zinit(i):
        zeros_v[pl.ds(i * 16, 16)] = zeros16

    plsc.parallel_loop(0, _SLICE // 16, unroll=8)(_zinit)

    off = pl.multiple_of(s * _SLICE, 8)
    pltpu.sync_copy(zeros_v, hist_sh.at[pl.ds(off, _SLICE)])
    plsc.subcore_barrier()

    @pl.when(w < _N_EXTRA)
    def _():
        eoff = pl.multiple_of(_EXTRA_BASE + w * _EXTRA, 128)
        pltpu.sync_copy(ei_hbm.at[pl.ds(0, 2), pl.ds(eoff, _EXTRA)], extra_v)

        def _erp(i):
            ebuf[pl.ds(i * 16, 16)] = extra_v[1, pl.ds(i * 16, 16)]

        plsc.parallel_loop(0, _EXTRA // 16, unroll=8)(_erp)
        pltpu.async_copy(
            ones_v.at[pl.ds(0, _EXTRA)], hist_sh.at[ebuf], e0, add=True
        )

    for j in range(_N_DESC):
        if j % _DESC_PER_STAGE == 0:
            k = j // _DESC_PER_STAGE
            pltpu.make_async_copy(
                ei_hbm.at[pl.ds(0, 2), pl.ds(base + k * _SCHUNK, _SCHUNK)],
                idx_v.at[pl.ds(0, 2), pl.ds(k * _SCHUNK, _SCHUNK)],
                stage_sems[k],
            ).wait()
        buf = bufs[j % _NBUF]
        sem = sems[j % _NBUF]
        if j >= _NBUF:
            pltpu.make_async_copy(ones_v, hist_sh.at[buf], sem).wait()

        def _rp(i, _buf=buf, _j=j):
            _buf[pl.ds(i * 16, 16)] = idx_v[1, pl.ds(_j * _CH + i * 16, 16)]

        plsc.parallel_loop(0, _CH // 16, unroll=8)(_rp)
        pltpu.async_copy(ones_v, hist_sh.at[buf], sem, add=True)

    for j in range(_N_DESC - _NBUF, _N_DESC):
        pltpu.make_async_copy(
            ones_v, hist_sh.at[bufs[j % _NBUF]], sems[j % _NBUF]
        ).wait()

    @pl.when(w < _N_EXTRA)
    def _():
        pltpu.make_async_copy(
            ones_v.at[pl.ds(0, _EXTRA)], hist_sh.at[ebuf], e0
        ).wait()

    plsc.subcore_barrier()

    oout = pl.multiple_of(c * _N_PAD + s * _SLICE, 8)
    pltpu.sync_copy(hist_sh.at[pl.ds(off, _SLICE)], sum_v)
    pltpu.sync_copy(sum_v, out_hbm.at[pl.ds(oout, _SLICE)])


@functools.cache
def _sc_hist():
    return pl.kernel(
        _sc_hist_body,
        out_type=jax.ShapeDtypeStruct((_NC * _N_PAD,), jnp.float32),
        mesh=plsc.VectorSubcoreMesh(core_axis_name="c", subcore_axis_name="s"),
        scratch_types=[
            pltpu.VMEM((2, _CHUNK), jnp.int32),
            pltpu.VMEM((2, _EXTRA), jnp.int32),
            pltpu.VMEM((_CH,), jnp.float32),
            pltpu.VMEM((_SLICE,), jnp.float32),
            pltpu.VMEM((_SLICE,), jnp.float32),
            pltpu.VMEM((_CH,), jnp.int32),
            pltpu.VMEM((_CH,), jnp.int32),
            pltpu.VMEM((_CH,), jnp.int32),
            pltpu.VMEM((_CH,), jnp.int32),
            pltpu.VMEM((_EXTRA,), jnp.int32),
            pltpu.MemorySpace.VMEM_SHARED((_N_PAD,), jnp.float32),
            pltpu.SemaphoreType.DMA,
            pltpu.SemaphoreType.DMA,
            pltpu.SemaphoreType.DMA,
            pltpu.SemaphoreType.DMA,
            pltpu.SemaphoreType.DMA,
            pltpu.SemaphoreType.DMA,
            pltpu.SemaphoreType.DMA,
            pltpu.SemaphoreType.DMA,
            pltpu.SemaphoreType.DMA,
            pltpu.SemaphoreType.DMA,
            pltpu.SemaphoreType.DMA,
        ],
    )


def _tc_finish_body(alpha_ref, part_ref, out_ref):
    h = part_ref[0] + part_ref[1]
    a = alpha_ref[0, 0]
    out_ref[...] = (jnp.log(h + 1.0) / _AVG_LOG_DEGREE) ** a


_tc_finish = pl.pallas_call(
    _tc_finish_body,
    out_shape=jax.ShapeDtypeStruct((_N_PAD // 128, 128), jnp.float32),
    in_specs=[
        pl.BlockSpec(memory_space=pltpu.MemorySpace.SMEM),
        pl.BlockSpec(memory_space=pltpu.MemorySpace.VMEM),
    ],
    out_specs=pl.BlockSpec(memory_space=pltpu.MemorySpace.VMEM),
)


def kernel(edge_index, alpha):
    partial = _sc_hist()(edge_index)
    part3d = partial.reshape(_NC, _N_PAD // 128, 128)
    s = _tc_finish(alpha.reshape(1, 1), part3d)
    return s.reshape(_N_PAD)[:_N_NODES]
